# Initial kernel scaffold; baseline (speedup 1.0000x reference)
import jax
import jax.numpy as jnp
from jax.experimental import pallas as pl


def kernel(x, table):
    raise NotImplementedError("write your pallas kernel here")


# speedup vs baseline: 2.7285x; 2.7285x over previous
import jax
import jax.numpy as jnp
from jax import lax
from jax.experimental import pallas as pl
from jax.experimental.pallas import tpu as pltpu, tpu_sc as plsc

_B, _S = 1024, 3200
_D = 16
_TOKEN_LEN = 16
_B_TOTAL = _B * _S
_NC, _NS = 2, 16
_NW = _NC * _NS
_LANES = 128
_K = 16
_CHUNK = _K * _LANES
_ROWS_TOTAL = _B_TOTAL // _LANES
_ROWS_PER_W = _ROWS_TOTAL // _NW
_N_CHUNKS = _ROWS_PER_W // _K


def _gather_body(xr_hbm, table_hbm, out_hbm, idx_v, rows_v, sem):
    wid = lax.axis_index("s") * _NC + lax.axis_index("c")
    row_base = wid * _ROWS_PER_W

    def step(c, carry):
        r0 = pl.multiple_of(row_base + c * _K, _K)
        pltpu.sync_copy(xr_hbm.at[pl.ds(r0, _K)], idx_v)
        copies = [
            pltpu.async_copy(table_hbm.at[idx_v.at[j]], rows_v.at[j], sem)
            for j in range(_K)
        ]
        for cp in copies:
            cp.wait()
        pltpu.sync_copy(rows_v, out_hbm.at[pl.ds(r0, _K)])
        return carry

    lax.fori_loop(0, _N_CHUNKS, step, 0)


def kernel(x, table):
    xr = x.reshape(_ROWS_TOTAL, _LANES)
    mesh = plsc.VectorSubcoreMesh(core_axis_name="c", subcore_axis_name="s")
    out = pl.kernel(
        _gather_body,
        out_type=jax.ShapeDtypeStruct((_ROWS_TOTAL, _LANES, _D), jnp.float32),
        mesh=mesh,
        scratch_types=[
            pltpu.VMEM((_K, _LANES), jnp.int32),
            pltpu.VMEM((_K, _LANES, _D), jnp.float32),
            pltpu.SemaphoreType.DMA,
        ],
        compiler_params=pltpu.CompilerParams(use_tc_tiling_on_sc=False),
    )(xr, table)
    return out.reshape(_B, _S // _TOKEN_LEN, _TOKEN_LEN * _D)


# speedup vs baseline: 2.7393x; 1.0040x over previous
import jax
import jax.numpy as jnp
from jax import lax
from jax.experimental import pallas as pl
from jax.experimental.pallas import tpu as pltpu, tpu_sc as plsc

_B, _S = 1024, 3200
_D = 16
_TOKEN_LEN = 16
_B_TOTAL = _B * _S
_NC, _NS = 2, 16
_NW = _NC * _NS
_LANES = 2048
_K = 1
_CHUNK = _K * _LANES
_ROWS_TOTAL = _B_TOTAL // _LANES
_ROWS_PER_W = _ROWS_TOTAL // _NW
_N_CHUNKS = _ROWS_PER_W // _K


def _gather_body(xr_hbm, table_hbm, out_hbm, idx_v, rows_v, sem):
    wid = lax.axis_index("s") * _NC + lax.axis_index("c")
    row_base = wid * _ROWS_PER_W

    def step(c, carry):
        r0 = pl.multiple_of(row_base + c * _K, _K)
        pltpu.sync_copy(xr_hbm.at[pl.ds(r0, _K)], idx_v)
        copies = [
            pltpu.async_copy(table_hbm.at[idx_v.at[j]], rows_v.at[j], sem)
            for j in range(_K)
        ]
        for cp in copies:
            cp.wait()
        pltpu.sync_copy(rows_v, out_hbm.at[pl.ds(r0, _K)])
        return carry

    lax.fori_loop(0, _N_CHUNKS, step, 0)


def kernel(x, table):
    xr = x.reshape(_ROWS_TOTAL, _LANES)
    mesh = plsc.VectorSubcoreMesh(core_axis_name="c", subcore_axis_name="s")
    out = pl.kernel(
        _gather_body,
        out_type=jax.ShapeDtypeStruct((_ROWS_TOTAL, _LANES, _D), jnp.float32),
        mesh=mesh,
        scratch_types=[
            pltpu.VMEM((_K, _LANES), jnp.int32),
            pltpu.VMEM((_K, _LANES, _D), jnp.float32),
            pltpu.SemaphoreType.DMA,
        ],
        compiler_params=pltpu.CompilerParams(use_tc_tiling_on_sc=False),
    )(xr, table)
    return out.reshape(_B, _S // _TOKEN_LEN, _TOKEN_LEN * _D)


# speedup vs baseline: 2.8678x; 1.0469x over previous
import jax
import jax.numpy as jnp
from jax import lax
from jax.experimental import pallas as pl
from jax.experimental.pallas import tpu as pltpu, tpu_sc as plsc

_B, _S = 1024, 3200
_D = 16
_TOKEN_LEN = 16
_B_TOTAL = _B * _S
_NC, _NS = 2, 16
_NW = _NC * _NS
_CHUNK = 2048
_ROWS_TOTAL = _B_TOTAL // _CHUNK
_N_CHUNKS = _ROWS_TOTAL // _NW


def _gather_body(xr_hbm, table_hbm, out_hbm, idx_v, rows_v,
                 si0, si1, sg0, sg1, so0, so1):
    wid = lax.axis_index("s") * _NC + lax.axis_index("c")
    base = wid * _N_CHUNKS
    si, sg, so = (si0, si1), (sg0, sg1), (so0, so1)

    def idx_start(c, b):
        pltpu.async_copy(xr_hbm.at[base + c], idx_v.at[b], si[b])

    def steady(c, b, first):
        pltpu.make_async_copy(xr_hbm.at[base + c], idx_v.at[b], si[b]).wait()
        if not first:
            pltpu.make_async_copy(rows_v.at[b], out_hbm.at[base + c],
                                  so[b]).wait()
        pltpu.async_copy(table_hbm.at[idx_v.at[b]], rows_v.at[b],
                         sg[b]).wait()
        pltpu.async_copy(rows_v.at[b], out_hbm.at[base + c], so[b])
        nxt = jnp.minimum(c + 2, _N_CHUNKS - 1)
        idx_start(nxt, b)

    idx_start(0, 0)
    idx_start(1, 1)
    steady(0, 0, first=True)
    steady(1, 1, first=True)

    def step(i, carry):
        steady(2 + 2 * i, 0, first=False)
        steady(3 + 2 * i, 1, first=False)
        return carry

    lax.fori_loop(0, (_N_CHUNKS - 2) // 2, step, 0)
    for b in (0, 1):
        pltpu.make_async_copy(rows_v.at[b], out_hbm.at[base], so[b]).wait()
        pltpu.make_async_copy(xr_hbm.at[base], idx_v.at[b], si[b]).wait()


def kernel(x, table):
    xr = x.reshape(_ROWS_TOTAL, _CHUNK)
    mesh = plsc.VectorSubcoreMesh(core_axis_name="c", subcore_axis_name="s")
    out = pl.kernel(
        _gather_body,
        out_type=jax.ShapeDtypeStruct((_ROWS_TOTAL, _CHUNK, _D), jnp.float32),
        mesh=mesh,
        scratch_types=[
            pltpu.VMEM((2, _CHUNK), jnp.int32),
            pltpu.VMEM((2, _CHUNK, _D), jnp.float32),
            pltpu.SemaphoreType.DMA,
            pltpu.SemaphoreType.DMA,
            pltpu.SemaphoreType.DMA,
            pltpu.SemaphoreType.DMA,
            pltpu.SemaphoreType.DMA,
            pltpu.SemaphoreType.DMA,
        ],
        compiler_params=pltpu.CompilerParams(use_tc_tiling_on_sc=False),
    )(xr, table)
    return out.reshape(_B, _S // _TOKEN_LEN, _TOKEN_LEN * _D)


# speedup vs baseline: 3.0289x; 1.0562x over previous
import jax
import jax.numpy as jnp
from jax import lax
from jax.experimental import pallas as pl
from jax.experimental.pallas import tpu as pltpu, tpu_sc as plsc

_B, _S = 1024, 3200
_D = 16
_VOCAB = 1000000
_VPAD = 1000064
_NC, _NS = 2, 16
_NW = _NC * _NS

_GW = 1024
_NG = _VOCAB // _GW
_TAIL = _VOCAB - _NG * _GW
_KMAX = 32

_XR, _XC = 1600, 2048
_RPW = _XR // _NW


def _worker_id():
    return lax.axis_index("s") * _NC + lax.axis_index("c")


def _transpose_rows(src, dst, n_rows, iota):
    def tr_block(j, carry):
        base = jnp.full((16,), j * 32, jnp.int32)
        for u in range(32):
            col = base + u
            row = plsc.load_gather(src, [iota, col])
            dst[pl.ds((j * 32 + u) * _D, _D)] = row
        return carry

    lax.fori_loop(0, n_rows // 32, tr_block, 0)


def _transpose_body(tt_hbm, tail_hbm, lin_hbm, in0, in1, out0, out1, tail_v,
                    si0, si1, so0, so1, st):
    wid = _worker_id()
    iota = lax.iota(jnp.int32, 16)
    ins, outs, si, so = (in0, in1), (out0, out1), (si0, si1), (so0, so1)

    def grp(k):
        return wid + k * _NW

    def in_copy(k, slot):
        return pltpu.make_async_copy(
            tt_hbm.at[:, pl.ds(grp(k) * _GW, _GW)], ins[slot], si[slot])

    def out_copy(k, slot):
        return pltpu.make_async_copy(
            outs[slot], lin_hbm.at[pl.ds(grp(k) * _GW * _D, _GW * _D)],
            so[slot])

    def steady(k, slot, first):
        g_ok = grp(k) < _NG

        @pl.when(g_ok)
        def _():
            in_copy(k, slot).wait()
            if not first:
                out_copy(k, slot).wait()
            _transpose_rows(ins[slot], outs[slot], _GW, iota)
            out_copy(k, slot).start()

        @pl.when(jnp.logical_and(g_ok, grp(k + 2) < _NG))
        def _():
            in_copy(k + 2, slot).start()

    in_copy(0, 0).start()
    in_copy(1, 1).start()
    steady(0, 0, first=True)
    steady(1, 1, first=True)

    def step(i, carry):
        steady(2 + 2 * i, 0, first=False)
        steady(3 + 2 * i, 1, first=False)
        return carry

    lax.fori_loop(0, (_KMAX - 2) // 2, step, 0)
    for slot in (0, 1):
        out_copy(slot, slot).wait()

    @pl.when(wid == 0)
    def _():
        v0 = _NG * _GW
        cp = pltpu.make_async_copy(tail_hbm, tail_v, st)
        cp.start()
        cp.wait()
        cp = pltpu.make_async_copy(
            tail_v, lin_hbm.at[pl.ds(v0 * _D, _TAIL * _D)], st)
        cp.start()
        cp.wait()


def _gather_body(xp_hbm, tab_hbm, out_hbm, idx0, idx1, rows0, rows1,
                 si0, si1, sg0, sg1, so0, so1):
    wid = _worker_id()
    base = wid * _RPW
    idxs, rows = (idx0, idx1), (rows0, rows1)
    si, sg, so = (si0, si1), (sg0, sg1), (so0, so1)

    def idx_copy(r, slot):
        return pltpu.make_async_copy(xp_hbm.at[base + r], idxs[slot], si[slot])

    def out_copy(r, slot):
        return pltpu.make_async_copy(rows[slot], out_hbm.at[base + r],
                                     so[slot])

    def steady(r, slot, first):
        idx_copy(r, slot).wait()
        if not first:
            out_copy(r, slot).wait()
        pltpu.async_copy(tab_hbm.at[idxs[slot]], rows[slot], sg[slot]).wait()
        out_copy(r, slot).start()
        nxt = jnp.minimum(r + 2, _RPW - 1)
        idx_copy(nxt, slot).start()

    idx_copy(0, 0).start()
    idx_copy(1, 1).start()
    steady(0, 0, first=True)
    steady(1, 1, first=True)

    def step(i, carry):
        steady(2 + 2 * i, 0, first=False)
        steady(3 + 2 * i, 1, first=False)
        return carry

    lax.fori_loop(0, (_RPW - 2) // 2, step, 0)
    for slot in (0, 1):
        out_copy(0, slot).wait()
        idx_copy(0, slot).wait()


def kernel(x, table):
    mesh = plsc.VectorSubcoreMesh(core_axis_name="c", subcore_axis_name="s")

    lin = pl.kernel(
        _transpose_body,
        out_type=jax.ShapeDtypeStruct((_VOCAB * _D,), jnp.float32),
        mesh=mesh,
        scratch_types=[
            pltpu.VMEM((16, _GW), jnp.float32),
            pltpu.VMEM((16, _GW), jnp.float32),
            pltpu.VMEM((_GW * _D,), jnp.float32),
            pltpu.VMEM((_GW * _D,), jnp.float32),
            pltpu.VMEM((_TAIL * _D,), jnp.float32),
            pltpu.SemaphoreType.DMA,
            pltpu.SemaphoreType.DMA,
            pltpu.SemaphoreType.DMA,
            pltpu.SemaphoreType.DMA,
            pltpu.SemaphoreType.DMA,
        ],
        compiler_params=pltpu.CompilerParams(needs_layout_passes=False),
    )(table.T, table[_NG * _GW:].reshape(_TAIL * _D))

    out3 = pl.kernel(
        _gather_body,
        out_type=jax.ShapeDtypeStruct((_XR, _XC, _D), jnp.float32),
        mesh=mesh,
        scratch_types=[
            pltpu.VMEM((_XC,), jnp.int32),
            pltpu.VMEM((_XC,), jnp.int32),
            pltpu.VMEM((_XC, _D), jnp.float32),
            pltpu.VMEM((_XC, _D), jnp.float32),
            pltpu.SemaphoreType.DMA,
            pltpu.SemaphoreType.DMA,
            pltpu.SemaphoreType.DMA,
            pltpu.SemaphoreType.DMA,
            pltpu.SemaphoreType.DMA,
            pltpu.SemaphoreType.DMA,
        ],
        compiler_params=pltpu.CompilerParams(use_tc_tiling_on_sc=False),
    )(x.reshape(_XR, _XC), lin.reshape(_VOCAB, _D))

    return out3.reshape(_B, _S // 16, 16 * _D)
